# baseline (speedup 1.0000x reference)
import functools

import jax
import jax.numpy as jnp
from jax import lax
from jax.experimental import pallas as pl
from jax.experimental.pallas import tpu as pltpu
from jax.experimental.pallas import tpu_sc as plsc

B, L, D = 4096, 50, 64
NC, NS = 2, 16
NW = NC * NS
CB = B // NW
SUB = 8
NSUB = CB // SUB
ROWS = SUB * L
LANES = 16


def _scores(sidx, uidx, pidx, nidx, dpos, dneg, ui_table, item_table):
    mesh = plsc.VectorSubcoreMesh(core_axis_name="c", subcore_axis_name="s")

    @functools.partial(
        pl.kernel,
        out_type=(
            jax.ShapeDtypeStruct((B,), jnp.float32),
            jax.ShapeDtypeStruct((B,), jnp.float32),
        ),
        mesh=mesh,
        compiler_params=pltpu.CompilerParams(needs_layout_passes=False,
                                             use_tc_tiling_on_sc=False),
        scratch_types=[
            pltpu.VMEM((CB * L,), jnp.int32),
            pltpu.VMEM((CB,), jnp.int32),
            pltpu.VMEM((CB,), jnp.int32),
            pltpu.VMEM((CB,), jnp.int32),
            pltpu.VMEM((CB,), jnp.float32),
            pltpu.VMEM((CB,), jnp.float32),
            pltpu.VMEM((ROWS, D), jnp.float32),
            pltpu.VMEM((ROWS, D), jnp.float32),
            pltpu.VMEM((CB, D), jnp.float32),
            pltpu.VMEM((CB, D), jnp.float32),
            pltpu.VMEM((CB, D), jnp.float32),
            pltpu.VMEM((CB, D), jnp.float32),
            pltpu.VMEM((CB,), jnp.float32),
            pltpu.VMEM((CB,), jnp.float32),
            pltpu.VMEM((1, D), jnp.float32),
            pltpu.SemaphoreType.DMA,
            pltpu.SemaphoreType.DMA,
            pltpu.SemaphoreType.DMA,
            pltpu.SemaphoreType.DMA,
            pltpu.SemaphoreType.DMA,
        ],
    )
    def k(sidx_hbm, uidx_hbm, pidx_hbm, nidx_hbm, dpos_hbm, dneg_hbm,
          ui_hbm, item_hbm, opos_hbm, oneg_hbm,
          sidx_v, uidx_v, pidx_v, nidx_v, dpos_v, dneg_v,
          stage0, stage1, ui_rows, pos_rows, neg_rows, acc_v,
          opos_v, oneg_v, t0_v, sem_u, sem_p, sem_n, sem_s0, sem_s1):
        wid = lax.axis_index("s") * NC + lax.axis_index("c")
        base = wid * CB

        pltpu.sync_copy(sidx_hbm.at[pl.ds(base * L, CB * L)], sidx_v)
        pltpu.sync_copy(uidx_hbm.at[pl.ds(base, CB)], uidx_v)
        pltpu.sync_copy(pidx_hbm.at[pl.ds(base, CB)], pidx_v)
        pltpu.sync_copy(nidx_hbm.at[pl.ds(base, CB)], nidx_v)
        pltpu.sync_copy(dpos_hbm.at[pl.ds(base, CB)], dpos_v)
        pltpu.sync_copy(dneg_hbm.at[pl.ds(base, CB)], dneg_v)
        pltpu.sync_copy(item_hbm.at[pl.ds(0, 1)], t0_v)

        cu = pltpu.async_copy(ui_hbm.at[uidx_v], ui_rows, sem_u)
        cp = pltpu.async_copy(item_hbm.at[pidx_v], pos_rows, sem_p)
        cn = pltpu.async_copy(item_hbm.at[nidx_v], neg_rows, sem_n)

        stages = (stage0, stage1)
        sems = (sem_s0, sem_s1)
        handles = [
            pltpu.async_copy(item_hbm.at[sidx_v.at[pl.ds(0, ROWS)]],
                             stage0, sem_s0),
            None,
        ]
        for s in range(NSUB):
            buf = s % 2
            if s + 1 < NSUB:
                nb = (s + 1) % 2
                handles[nb] = pltpu.async_copy(
                    item_hbm.at[sidx_v.at[pl.ds((s + 1) * ROWS, ROWS)]],
                    stages[nb], sems[nb])
            handles[buf].wait()
            st = stages[buf]

            def outer(bl, _, s=s, st=st):
                b = s * SUB + bl

                def inner(l, carry, bl=bl, st=st):
                    a0, a1, a2, a3 = carry
                    r = bl * L + l
                    return (a0 + st[r, pl.ds(0, LANES)],
                            a1 + st[r, pl.ds(LANES, LANES)],
                            a2 + st[r, pl.ds(2 * LANES, LANES)],
                            a3 + st[r, pl.ds(3 * LANES, LANES)])

                z = jnp.zeros((LANES,), jnp.float32)
                a0, a1, a2, a3 = lax.fori_loop(0, L, inner, (z, z, z, z))
                acc_v[b, pl.ds(0, LANES)] = a0
                acc_v[b, pl.ds(LANES, LANES)] = a1
                acc_v[b, pl.ds(2 * LANES, LANES)] = a2
                acc_v[b, pl.ds(3 * LANES, LANES)] = a3
                return 0

            lax.fori_loop(0, SUB, outer, 0)

        cu.wait()
        cp.wait()
        cn.wait()

        iota = lax.iota(jnp.int32, LANES)
        for g in range(CB // LANES):
            rbase = g * LANES
            rows_idx = iota + rbase

            def cbody(l, n0):
                sv = plsc.load_gather(sidx_v, [rbase * L + iota * L + l])
                return n0 + jnp.where(sv == 0, jnp.float32(1), jnp.float32(0))

            n0 = lax.fori_loop(0, L, cbody, jnp.zeros((LANES,), jnp.float32))
            inv = jnp.float32(1) / (jnp.float32(L) - n0 + jnp.float32(1e-9))

            zrow = jnp.zeros((LANES,), jnp.int32)

            def dbody(d, carry):
                ps, ns = carry
                dcol = jnp.full((LANES,), d, jnp.int32)
                t0c = plsc.load_gather(t0_v, [zrow, dcol])
                a = plsc.load_gather(acc_v, [rows_idx, dcol]) - n0 * t0c
                u = plsc.load_gather(ui_rows, [rows_idx, dcol]) + a * inv
                p = plsc.load_gather(pos_rows, [rows_idx, dcol])
                nn = plsc.load_gather(neg_rows, [rows_idx, dcol])
                return (ps + u * p, ns + u * nn)

            z = jnp.zeros((LANES,), jnp.float32)
            ps, ns = lax.fori_loop(0, D, dbody, (z, z))
            opos_v[pl.ds(rbase, LANES)] = ps - dpos_v[pl.ds(rbase, LANES)]
            oneg_v[pl.ds(rbase, LANES)] = ns - dneg_v[pl.ds(rbase, LANES)]

        pltpu.sync_copy(opos_v, opos_hbm.at[pl.ds(base, CB)])
        pltpu.sync_copy(oneg_v, oneg_hbm.at[pl.ds(base, CB)])

    return k(sidx, uidx, pidx, nidx, dpos, dneg, ui_table, item_table)


def kernel(user_inputs, seq_inputs, pos_inputs, neg_inputs, distance_pos,
           distance_neg, ui_table, item_table):
    sidx = seq_inputs.reshape(-1).astype(jnp.int32)
    uidx = user_inputs.reshape(-1).astype(jnp.int32)
    pidx = pos_inputs.reshape(-1).astype(jnp.int32)
    nidx = neg_inputs.reshape(-1).astype(jnp.int32)
    dpos = distance_pos.reshape(-1).astype(jnp.float32)
    dneg = distance_neg.reshape(-1).astype(jnp.float32)
    pos_s, neg_s = _scores(sidx, uidx, pidx, nidx, dpos, dneg,
                           ui_table, item_table)
    return (pos_s.reshape(B, 1), neg_s.reshape(B, 1))
